# Initial kernel scaffold; baseline (speedup 1.0000x reference)
import jax
import jax.numpy as jnp
from jax.experimental import pallas as pl


def kernel(x, edge_index, W1, b1, W2, b2, W_lin, b_lin):
    raise NotImplementedError("write your pallas kernel here")


# speedup vs baseline: 13.1570x; 13.1570x over previous
import functools

import jax
import jax.numpy as jnp
from jax import lax
from jax.experimental import pallas as pl
from jax.experimental.pallas import tpu as pltpu
from jax.experimental.pallas import tpu_sc as plsc

NC = 2
NS = 16
NW = NC * NS
LANES = 16

CHUNK = 80
ZROWS = 128


def _mesh():
    return plsc.VectorSubcoreMesh(
        core_axis_name="c", subcore_axis_name="s", num_cores=NC, num_subcores=NS
    )


def _make_deg_kernel(E, NPAD):
    e_per = E // NW
    n_chunks = e_per // CHUNK
    rows_per_tile = NPAD // NS

    @functools.partial(
        pl.kernel,
        out_type=jax.ShapeDtypeStruct((NC, NPAD), jnp.float32),
        mesh=_mesh(),
        scratch_types=[
            pltpu.VMEM((CHUNK,), jnp.int32),
            pltpu.VMEM((CHUNK,), jnp.float32),
            pltpu.VMEM((rows_per_tile,), jnp.float32),
            pltpu.VMEM_SHARED((NPAD,), jnp.float32),
        ],
    )
    def deg_kernel(dst_hbm, out_hbm, idx_v, ones_v, zero_v, acc_sh):
        cid = lax.axis_index("c")
        sid = lax.axis_index("s")
        wid = sid * NC + cid

        def fill_ones(i, carry):
            ones_v[pl.ds(i * LANES, LANES)] = jnp.full((LANES,), 1.0, jnp.float32)
            return carry

        lax.fori_loop(0, CHUNK // LANES, fill_ones, 0)

        def fill_zero(i, carry):
            zero_v[pl.ds(i * LANES, LANES)] = jnp.zeros((LANES,), jnp.float32)
            return carry

        lax.fori_loop(0, rows_per_tile // LANES, fill_zero, 0)

        r0 = sid * rows_per_tile
        pltpu.sync_copy(zero_v, acc_sh.at[pl.ds(r0, rows_per_tile)])
        plsc.subcore_barrier()

        def body(j, carry):
            base = wid * e_per + j * CHUNK
            pltpu.sync_copy(dst_hbm.at[pl.ds(base, CHUNK)], idx_v)
            pltpu.sync_copy(ones_v, acc_sh.at[idx_v], add=True)
            return carry

        lax.fori_loop(0, n_chunks, body, 0)

        plsc.subcore_barrier()
        pltpu.sync_copy(
            acc_sh.at[pl.ds(r0, rows_per_tile)],
            out_hbm.at[cid, pl.ds(r0, rows_per_tile)],
        )

    return deg_kernel


def _make_agg_kernel(D, E, NPAD):
    e_per = E // NW
    n_chunks = e_per // CHUNK
    rows_per_tile = NPAD // NS

    @functools.partial(
        pl.kernel,
        out_type=jax.ShapeDtypeStruct((NC, NPAD, D), jnp.float32),
        mesh=_mesh(),
        scratch_types=[
            pltpu.VMEM((CHUNK,), jnp.int32),
            pltpu.VMEM((CHUNK,), jnp.int32),
            pltpu.VMEM((CHUNK, D), jnp.float32),
            pltpu.VMEM((ZROWS, D), jnp.float32),
            pltpu.VMEM_SHARED((NPAD, D), jnp.float32),
            pltpu.SemaphoreType.DMA,
        ],
    )
    def agg_kernel(g_hbm, src_hbm, dst_hbm, out_hbm,
                   src_v, dst_v, rows_v, zero_v, acc_sh, sem):
        cid = lax.axis_index("c")
        sid = lax.axis_index("s")
        wid = sid * NC + cid

        def fill_zero(i, carry):
            zero_v[i // (D // LANES), pl.ds((i % (D // LANES)) * LANES, LANES)] = (
                jnp.zeros((LANES,), jnp.float32)
            )
            return carry

        lax.fori_loop(0, ZROWS * (D // LANES), fill_zero, 0)

        r0 = sid * rows_per_tile
        for k in range(rows_per_tile // ZROWS):
            pltpu.sync_copy(zero_v, acc_sh.at[pl.ds(r0 + k * ZROWS, ZROWS)])
        plsc.subcore_barrier()

        def body(j, carry):
            base = wid * e_per + j * CHUNK
            pltpu.sync_copy(src_hbm.at[pl.ds(base, CHUNK)], src_v)
            pltpu.sync_copy(dst_hbm.at[pl.ds(base, CHUNK)], dst_v)
            pltpu.async_copy(g_hbm.at[src_v], rows_v, sem).wait()
            pltpu.sync_copy(rows_v, acc_sh.at[dst_v], add=True)
            return carry

        lax.fori_loop(0, n_chunks, body, 0)

        plsc.subcore_barrier()
        pltpu.sync_copy(
            acc_sh.at[pl.ds(r0, rows_per_tile)],
            out_hbm.at[cid, pl.ds(r0, rows_per_tile)],
        )

    return agg_kernel


def _tc_first(x, W, degT, BN):
    Nn, D = x.shape

    def body(x_ref, w_ref, deg_ref, o_ref):
        deg = deg_ref[:, 0:1] + deg_ref[:, 1:2] + 1.0
        dinv = lax.rsqrt(deg)
        h = jnp.dot(x_ref[...], w_ref[...], preferred_element_type=jnp.float32)
        o_ref[...] = h * dinv

    return pl.pallas_call(
        body,
        grid=(Nn // BN,),
        in_specs=[
            pl.BlockSpec((BN, D), lambda i: (i, 0)),
            pl.BlockSpec((D, D), lambda i: (0, 0)),
            pl.BlockSpec((BN, 2), lambda i: (i, 0)),
        ],
        out_specs=pl.BlockSpec((BN, D), lambda i: (i, 0)),
        out_shape=jax.ShapeDtypeStruct((Nn, D), jnp.float32),
    )(x, W, degT)


def _tc_next(part, g_prev, degT, b, W, BN, final, b_out=None):
    Nn, D = g_prev.shape

    def body(p_ref, g_ref, deg_ref, b_ref, w_ref, bo_ref, o_ref):
        deg = deg_ref[:, 0:1] + deg_ref[:, 1:2] + 1.0
        dinv = lax.rsqrt(deg)
        agg = p_ref[0] + p_ref[1] + g_ref[...]
        h = jnp.maximum(agg * dinv + b_ref[...], 0.0)
        hw = jnp.dot(h, w_ref[...], preferred_element_type=jnp.float32)
        if final:
            o_ref[...] = hw + bo_ref[...]
        else:
            o_ref[...] = hw * dinv

    if b_out is None:
        b_out = jnp.zeros((1, D), jnp.float32)

    return pl.pallas_call(
        body,
        grid=(Nn // BN,),
        in_specs=[
            pl.BlockSpec((2, BN, D), lambda i: (0, i, 0)),
            pl.BlockSpec((BN, D), lambda i: (i, 0)),
            pl.BlockSpec((BN, 2), lambda i: (i, 0)),
            pl.BlockSpec((1, D), lambda i: (0, 0)),
            pl.BlockSpec((D, D), lambda i: (0, 0)),
            pl.BlockSpec((1, D), lambda i: (0, 0)),
        ],
        out_specs=pl.BlockSpec((BN, D), lambda i: (i, 0)),
        out_shape=jax.ShapeDtypeStruct((Nn, D), jnp.float32),
    )(part, g_prev, degT, b, W, b_out)


@jax.jit
def kernel(x, edge_index, W1, b1, W2, b2, W_lin, b_lin):
    Nn, D = x.shape
    E = edge_index.shape[1]
    NPAD = ((Nn + NW * LANES - 1) // (NW * LANES)) * (NW * LANES)
    BN = 1000

    src = edge_index[0]
    dst = edge_index[1]

    deg_parts = _make_deg_kernel(E, NPAD)(dst)
    degT = deg_parts.T

    g1 = _tc_first(x, W1, degT, BN)
    part1 = _make_agg_kernel(D, E, NPAD)(g1, src, dst)

    g2 = _tc_next(part1, g1, degT, b1.reshape(1, D), W2, BN, final=False)
    part2 = _make_agg_kernel(D, E, NPAD)(g2, src, dst)

    y = _tc_next(part2, g2, degT, b2.reshape(1, D), W_lin, BN,
                 final=True, b_out=b_lin.reshape(1, D))
    return y
